# Initial kernel scaffold; baseline (speedup 1.0000x reference)
import jax
import jax.numpy as jnp
from jax.experimental import pallas as pl


def kernel(x, edge_index, e, volume_id, dec0_W0, dec0_b0, dec0_g0, dec0_beta0, dec0_W1, dec0_b1, dec0_g1, dec0_beta1, dec0_W2, dec0_b2, dec1_W0, dec1_b0, dec1_g0, dec1_beta0, dec1_W1, dec1_b1, dec1_g1, dec1_beta1, dec1_W2, dec1_b2, dec2_W0, dec2_b0, dec2_g0, dec2_beta0, dec2_W1, dec2_b1, dec2_g1, dec2_beta1, dec2_W2, dec2_b2):
    raise NotImplementedError("write your pallas kernel here")


# speedup vs baseline: 5.2269x; 5.2269x over previous
import functools

import jax
import jax.numpy as jnp
from jax import lax
from jax.experimental import pallas as pl
from jax.experimental.pallas import tpu as pltpu
from jax.experimental.pallas import tpu_sc as plsc

E = 320000
NN = 10000
H = 128
NC = 2
NS = 16
NW = NC * NS
EPW = E // NW
SUB = 80
NSUB = EPW // SUB
TILE = 1600
NTILE = E // TILE


def _sc_gather_body(x_hbm, s_hbm, t_hbm, vol_hbm, xs_hbm, xt_hbm, dec_hbm,
                    idx_s, idx_t, vol_v, dec_v, rows_s, rows_t, sem_s, sem_t):
    wid = lax.axis_index("s") * NC + lax.axis_index("c")
    base = wid * EPW
    pltpu.sync_copy(vol_hbm, vol_v)
    pltpu.sync_copy(s_hbm.at[pl.ds(base, EPW)], idx_s)
    pltpu.sync_copy(t_hbm.at[pl.ds(base, EPW)], idx_t)

    def dec_body(i, carry):
        si = idx_s[pl.ds(i * 16, 16)]
        ti = idx_t[pl.ds(i * 16, 16)]
        vs = plsc.load_gather(vol_v, [si])
        ve = plsc.load_gather(vol_v, [ti])
        two = jnp.full((16,), 2, jnp.int32)
        one = jnp.full((16,), 1, jnp.int32)
        zero = jnp.full((16,), 0, jnp.int32)
        d = jnp.where(ve >= two, jnp.where(vs >= two, two, one), zero)
        dec_v[pl.ds(i * 16, 16)] = d
        return carry

    lax.fori_loop(0, EPW // 16, dec_body, 0)
    pltpu.sync_copy(dec_v, dec_hbm.at[pl.ds(base, EPW)])

    def sub_body(j, carry):
        off = j * SUB
        cp_s = pltpu.async_copy(x_hbm.at[idx_s.at[pl.ds(off, SUB)]], rows_s, sem_s)
        cp_t = pltpu.async_copy(x_hbm.at[idx_t.at[pl.ds(off, SUB)]], rows_t, sem_t)
        cp_s.wait()
        pltpu.sync_copy(rows_s, xs_hbm.at[pl.ds(base + off, SUB)])
        cp_t.wait()
        pltpu.sync_copy(rows_t, xt_hbm.at[pl.ds(base + off, SUB)])
        return carry

    lax.fori_loop(0, NSUB, sub_body, 0)


@functools.cache
def _sc_gather():
    return pl.kernel(
        _sc_gather_body,
        out_type=[
            jax.ShapeDtypeStruct((E, H), jnp.float32),
            jax.ShapeDtypeStruct((E, H), jnp.float32),
            jax.ShapeDtypeStruct((E,), jnp.int32),
        ],
        mesh=plsc.VectorSubcoreMesh(core_axis_name="c", subcore_axis_name="s"),
        compiler_params=pltpu.CompilerParams(needs_layout_passes=False),
        scratch_types=[
            pltpu.VMEM((EPW,), jnp.int32),
            pltpu.VMEM((EPW,), jnp.int32),
            pltpu.VMEM((NN,), jnp.int32),
            pltpu.VMEM((EPW,), jnp.int32),
            pltpu.VMEM((SUB, H), jnp.float32),
            pltpu.VMEM((SUB, H), jnp.float32),
            pltpu.SemaphoreType.DMA,
            pltpu.SemaphoreType.DMA,
        ],
    )


def _ln_relu(h, g, b):
    mu = jnp.mean(h, axis=-1, keepdims=True)
    hc = h - mu
    var = jnp.mean(hc * hc, axis=-1, keepdims=True)
    return jnp.maximum(hc * lax.rsqrt(var + 1e-5) * g + b, 0.0)


def _tc_body(dec_ref, xs_ref, xt_ref, e_ref, w0a_ref, w0b_ref, w0c_ref,
             b0_ref, g0_ref, bt0_ref, w1_ref, b1_ref, g1_ref, bt1_ref,
             w2_ref, b2_ref, out_ref):
    xs = xs_ref[...]
    xt = xt_ref[...]
    ev = e_ref[...]
    dec = dec_ref[...]
    outs = []
    for d in range(3):
        h = (jnp.dot(xs, w0a_ref[d], preferred_element_type=jnp.float32)
             + jnp.dot(xt, w0b_ref[d], preferred_element_type=jnp.float32)
             + jnp.dot(ev, w0c_ref[d], preferred_element_type=jnp.float32)
             + b0_ref[d])
        h = _ln_relu(h, g0_ref[d], bt0_ref[d])
        h = jnp.dot(h, w1_ref[d], preferred_element_type=jnp.float32) + b1_ref[d]
        h = _ln_relu(h, g1_ref[d], bt1_ref[d])
        o = jnp.dot(h, w2_ref[d], preferred_element_type=jnp.float32) + b2_ref[d]
        outs.append(o)
    out_ref[...] = jnp.where(dec == 2, outs[2],
                             jnp.where(dec == 1, outs[1], outs[0]))


def _edge_spec(width):
    return pl.BlockSpec((TILE, width), lambda i: (i, 0))


def _full_spec(shape):
    return pl.BlockSpec(shape, lambda i: (0,) * len(shape))


_tc_mlp = pl.pallas_call(
    _tc_body,
    grid=(NTILE,),
    in_specs=[
        _edge_spec(1),
        _edge_spec(H),
        _edge_spec(H),
        _edge_spec(H),
        _full_spec((3, H, H)),
        _full_spec((3, H, H)),
        _full_spec((3, H, H)),
        _full_spec((3, 1, H)),
        _full_spec((3, 1, H)),
        _full_spec((3, 1, H)),
        _full_spec((3, H, H)),
        _full_spec((3, 1, H)),
        _full_spec((3, 1, H)),
        _full_spec((3, 1, H)),
        _full_spec((3, H, 1)),
        _full_spec((3, 1, 1)),
    ],
    out_specs=_edge_spec(1),
    out_shape=jax.ShapeDtypeStruct((E, 1), jnp.float32),
    compiler_params=pltpu.CompilerParams(
        dimension_semantics=("arbitrary",),
    ),
)


def kernel(x, edge_index, e, volume_id,
           dec0_W0, dec0_b0, dec0_g0, dec0_beta0, dec0_W1, dec0_b1, dec0_g1,
           dec0_beta1, dec0_W2, dec0_b2,
           dec1_W0, dec1_b0, dec1_g0, dec1_beta0, dec1_W1, dec1_b1, dec1_g1,
           dec1_beta1, dec1_W2, dec1_b2,
           dec2_W0, dec2_b0, dec2_g0, dec2_beta0, dec2_W1, dec2_b1, dec2_g1,
           dec2_beta1, dec2_W2, dec2_b2):
    s = edge_index[0]
    t = edge_index[1]
    xs, xt, dec = _sc_gather()(x, s, t, volume_id)

    w0 = jnp.stack([dec0_W0, dec1_W0, dec2_W0])
    w0a = w0[:, :H, :]
    w0b = w0[:, H:2 * H, :]
    w0c = w0[:, 2 * H:, :]
    b0 = jnp.stack([dec0_b0, dec1_b0, dec2_b0])[:, None, :]
    g0 = jnp.stack([dec0_g0, dec1_g0, dec2_g0])[:, None, :]
    bt0 = jnp.stack([dec0_beta0, dec1_beta0, dec2_beta0])[:, None, :]
    w1 = jnp.stack([dec0_W1, dec1_W1, dec2_W1])
    b1 = jnp.stack([dec0_b1, dec1_b1, dec2_b1])[:, None, :]
    g1 = jnp.stack([dec0_g1, dec1_g1, dec2_g1])[:, None, :]
    bt1 = jnp.stack([dec0_beta1, dec1_beta1, dec2_beta1])[:, None, :]
    w2 = jnp.stack([dec0_W2, dec1_W2, dec2_W2])
    b2 = jnp.stack([dec0_b2, dec1_b2, dec2_b2])[:, None, :]

    return _tc_mlp(dec.reshape(E, 1), xs, xt, e,
                   w0a, w0b, w0c, b0, g0, bt0, w1, b1, g1, bt1, w2, b2)
